# Initial kernel scaffold; baseline (speedup 1.0000x reference)
import functools

import jax
import jax.numpy as jnp
from jax import lax
from jax.experimental import pallas as pl
from jax.experimental.pallas import tpu as pltpu
from jax.experimental.pallas import tpu_sc as plsc

N = 10000
NPAD = 10240
E = 320000
D = 128
NUM_ET = 40
ET_PAD = 64

L = 16
NC = 2
NS = 16
NW = NC * NS
EPT = E // NW

C = 80
NCHUNK = EPT // C

_SELU_LAM = 1.0507009873554804934193349852946
_SELU_ALPHA = 1.6732632423543772848170429916717

_mesh = plsc.VectorSubcoreMesh(core_axis_name="c", subcore_axis_name="s")


def _wid():
    return lax.axis_index("s") * NC + lax.axis_index("c")


def _tc_proj_body(x_ref, w_ref, a2_ref, wr_ref, rel_ref, ae_ref,
                  h_ref, s_ref, re_ref):
    h = jnp.dot(x_ref[...], w_ref[...], preferred_element_type=jnp.float32)
    h_ref[...] = h
    s_ref[...] = jnp.dot(h, a2_ref[...], preferred_element_type=jnp.float32)
    rp = jnp.dot(rel_ref[...], wr_ref[...], preferred_element_type=jnp.float32)
    re_ref[...] = jnp.dot(rp, ae_ref[...], preferred_element_type=jnp.float32)


_tc_proj = pl.pallas_call(
    _tc_proj_body,
    out_shape=[
        jax.ShapeDtypeStruct((N, D), jnp.float32),
        jax.ShapeDtypeStruct((N, 8), jnp.float32),
        jax.ShapeDtypeStruct((NUM_ET, 8), jnp.float32),
    ],
)


@functools.partial(
    pl.kernel,
    out_type=[
        jax.ShapeDtypeStruct((E,), jnp.float32),
        jax.ShapeDtypeStruct((NW, NPAD), jnp.float32),
    ],
    mesh=_mesh,
    scratch_types=[
        pltpu.VMEM((EPT,), jnp.int32),
        pltpu.VMEM((EPT,), jnp.int32),
        pltpu.VMEM((EPT,), jnp.int32),
        pltpu.VMEM((N,), jnp.float32),
        pltpu.VMEM((N,), jnp.float32),
        pltpu.VMEM((ET_PAD,), jnp.float32),
        pltpu.VMEM((EPT,), jnp.float32),
        pltpu.VMEM((NPAD,), jnp.float32),
    ],
)
def _sc_scores(src_hbm, dst_hbm, et_hbm, hl_hbm, hr_hbm, re_hbm,
               ex_hbm, psum_hbm,
               src_v, dst_v, et_v, hl_v, hr_v, re_v, ex_v, ns_v):
    w = _wid()
    base = pl.multiple_of(w * EPT, 8)
    pltpu.sync_copy(src_hbm.at[pl.ds(base, EPT)], src_v)
    pltpu.sync_copy(dst_hbm.at[pl.ds(base, EPT)], dst_v)
    pltpu.sync_copy(et_hbm.at[pl.ds(base, EPT)], et_v)
    pltpu.sync_copy(hl_hbm, hl_v)
    pltpu.sync_copy(hr_hbm, hr_v)
    pltpu.sync_copy(re_hbm, re_v)

    def zero_body(i, carry):
        ns_v[pl.ds(i * L, L)] = jnp.zeros((L,), jnp.float32)
        return carry

    lax.fori_loop(0, NPAD // L, zero_body, 0)

    def body(i, carry):
        sl = pl.ds(i * L, L)
        sv = src_v[sl]
        dv = dst_v[sl]
        ev = et_v[sl]
        gl = plsc.load_gather(hl_v, [sv])
        gr = plsc.load_gather(hr_v, [dv])
        ge = plsc.load_gather(re_v, [ev])
        z = gl + gr + ge
        e = jnp.where(z >= 0.0, z, 0.2 * z)
        ex = jnp.exp(e)
        ex_v[sl] = ex
        plsc.addupdate_scatter(ns_v, [dv], ex)
        return carry

    lax.fori_loop(0, EPT // L, body, 0)

    pltpu.sync_copy(ex_v, ex_hbm.at[pl.ds(base, EPT)])
    pltpu.sync_copy(ns_v, psum_hbm.at[w])


SLC = NPAD // NW


@functools.partial(
    pl.kernel,
    out_type=jax.ShapeDtypeStruct((NPAD,), jnp.float32),
    mesh=_mesh,
    scratch_types=[
        pltpu.VMEM((NW, SLC), jnp.float32),
        pltpu.VMEM((SLC,), jnp.float32),
    ],
)
def _sc_reduce(psum_hbm, ssum_hbm, buf_v, acc_v):
    w = _wid()
    off = pl.multiple_of(w * SLC, 8)
    pltpu.sync_copy(psum_hbm.at[:, pl.ds(off, SLC)], buf_v)

    def body(j, carry):
        sl = pl.ds(j * L, L)
        acc = jnp.zeros((L,), jnp.float32)
        for r in range(NW):
            acc = acc + buf_v[r, sl]
        acc_v[sl] = acc
        return carry

    lax.fori_loop(0, SLC // L, body, 0)
    pltpu.sync_copy(acc_v, ssum_hbm.at[pl.ds(off, SLC)])


RPT = NPAD // NS


@functools.partial(
    pl.kernel,
    out_type=jax.ShapeDtypeStruct((NC, NPAD, D), jnp.float32),
    mesh=_mesh,
    scratch_types=[
        pltpu.VMEM((EPT,), jnp.int32),
        pltpu.VMEM((NCHUNK, C), jnp.int32),
        pltpu.VMEM((EPT,), jnp.float32),
        pltpu.VMEM((NPAD,), jnp.float32),
        pltpu.VMEM((C,), jnp.float32),
        pltpu.VMEM((C, D), jnp.float32),
        pltpu.VMEM((C, D), jnp.float32),
        pltpu.VMEM((C, D), jnp.float32),
        pltpu.VMEM((C, D), jnp.float32),
        pltpu.VMEM_SHARED((NPAD, D), jnp.float32),
        pltpu.SemaphoreType.DMA,
        pltpu.SemaphoreType.DMA,
        pltpu.SemaphoreType.DMA,
        pltpu.SemaphoreType.DMA,
    ],
)
def _sc_msg(src_hbm, dstr_hbm, ex_hbm, ssum_hbm, h_hbm, agg_hbm,
            src_v, dst2_v, ex_v, ssum_v, alpha_v,
            rows0_v, rows1_v, sb0_v, sb1_v, agg_sh,
            sem_g0, sem_g1, sem_s0, sem_s1):
    cid = lax.axis_index("c")
    sid = lax.axis_index("s")
    w = sid * NC + cid
    base = pl.multiple_of(w * EPT, 8)

    rows_bufs = (rows0_v, rows1_v)
    sbufs = (sb0_v, sb1_v)
    gsems = (sem_g0, sem_g1)
    ssems = (sem_s0, sem_s1)

    pltpu.sync_copy(src_hbm.at[pl.ds(base, EPT)], src_v)
    pltpu.sync_copy(dstr_hbm.at[w], dst2_v)
    pltpu.sync_copy(ex_hbm.at[pl.ds(base, EPT)], ex_v)
    pltpu.sync_copy(ssum_hbm, ssum_v)

    def zrow(i, carry):
        rows0_v[i // (D // L), pl.ds((i % (D // L)) * L, L)] = (
            jnp.zeros((L,), jnp.float32))
        return carry

    lax.fori_loop(0, C * D // L, zrow, 0)
    rbase = pl.multiple_of(sid * RPT, 8)
    for t in range(RPT // C):
        pltpu.sync_copy(rows0_v, agg_sh.at[pl.ds(rbase + t * C, C), :])
    plsc.subcore_barrier()

    def start_gather(c, b):
        pltpu.async_copy(
            h_hbm.at[src_v.at[pl.ds(c * C, C)]], rows_bufs[b], gsems[b])

    def start_scatter(c, b):
        pltpu.async_copy(
            sbufs[b], agg_sh.at[dst2_v.at[c]], ssems[b], add=True)

    def wait_gather(b):
        pltpu.make_async_copy(
            h_hbm.at[src_v.at[pl.ds(0, C)]], rows_bufs[b], gsems[b]).wait()

    def wait_scatter(b):
        pltpu.make_async_copy(
            sbufs[b], agg_sh.at[dst2_v.at[0]], ssems[b]).wait()

    def compute(c, b):
        rows = rows_bufs[b]
        sb = sbufs[b]
        for j in range(C // L):
            dv = dst2_v[c, pl.ds(j * L, L)]
            s = plsc.load_gather(ssum_v, [dv])
            al = ex_v[pl.ds(c * C + j * L, L)] / (s + 1e-16)
            alpha_v[pl.ds(j * L, L)] = al

        def ebody(e, carry):
            av = lax.broadcast(alpha_v[e], (L,))
            for k in range(D // L):
                sl = pl.ds(k * L, L)
                sb[e, sl] = rows[e, sl] * av
            return carry

        lax.fori_loop(0, C, ebody, 0)

    start_gather(0, 0)
    start_gather(1, 1)

    def gbody(g, carry):
        for b in range(2):
            c = 2 * g + b
            wait_gather(b)

            @pl.when(g >= 1)
            def _():
                wait_scatter(b)

            compute(c, b)

            @pl.when(c + 2 < NCHUNK)
            def _():
                start_gather(c + 2, b)

            start_scatter(c, b)
        return carry

    lax.fori_loop(0, (NCHUNK - 1) // 2, gbody, 0)

    ct = NCHUNK - 1
    wait_gather(0)
    wait_scatter(0)
    compute(ct, 0)
    start_scatter(ct, 0)
    wait_scatter(1)
    wait_scatter(0)

    plsc.subcore_barrier()
    pltpu.sync_copy(agg_sh.at[pl.ds(rbase, RPT), :],
                    agg_hbm.at[cid, pl.ds(rbase, RPT), :])


def _tc_selu_body(a_ref, o_ref):
    x = a_ref[0] + a_ref[1]
    o_ref[...] = _SELU_LAM * jnp.where(
        x > 0.0, x, _SELU_ALPHA * jnp.expm1(x))


_tc_selu = pl.pallas_call(
    _tc_selu_body,
    out_shape=jax.ShapeDtypeStruct((NPAD, D), jnp.float32),
)


def kernel(node_features, edge_index, edge_type, W, W_r, rel_emb,
           a_l, a_r, a_e):
    src = edge_index[0].astype(jnp.int32)
    dst = edge_index[1].astype(jnp.int32)
    et = edge_type.astype(jnp.int32)
    a2 = jnp.pad(jnp.stack([a_l, a_r], axis=1), ((0, 0), (0, 6)))
    ae8 = jnp.pad(a_e[:, None], ((0, 0), (0, 7)))

    h, s8, re8 = _tc_proj(node_features, W, a2, W_r, rel_emb, ae8)
    hl = s8[:, 0]
    hr = s8[:, 1]
    re64 = jnp.pad(re8[:, 0], (0, ET_PAD - NUM_ET))

    ex, psum = _sc_scores(src, dst, et, hl, hr, re64)
    ssum = _sc_reduce(psum)
    dst_r = dst.reshape(NW, NCHUNK, C)
    agg2 = _sc_msg(src, dst_r, ex, ssum, h)
    out = _tc_selu(agg2)
    return out[:N]


# speedup vs baseline: 24.9289x; 24.9289x over previous
import functools

import jax
import jax.numpy as jnp
from jax import lax
from jax.experimental import pallas as pl
from jax.experimental.pallas import tpu as pltpu
from jax.experimental.pallas import tpu_sc as plsc

N = 10000
NPAD = 10240
E = 320000
D = 128
DH = 64
NUM_ET = 40
ET_PAD = 128

L = 16
NC = 2
NS = 16
NW = NC * NS
EPT = E // NW

C = 80
NCHUNK = EPT // C
VPC = C // L

SLC = NPAD // NS

_SELU_LAM = 1.0507009873554804934193349852946
_SELU_ALPHA = 1.6732632423543772848170429916717

_mesh = plsc.VectorSubcoreMesh(core_axis_name="c", subcore_axis_name="s")
_sc_params = pltpu.CompilerParams(
    needs_layout_passes=False, use_tc_tiling_on_sc=False)


def _tc_proj_body(x_ref, w_ref, a2_ref, wr_ref, rel_ref, ae_ref,
                  ha_ref, hb_ref, s_ref, re_ref):
    h = jnp.dot(x_ref[...], w_ref[...], preferred_element_type=jnp.float32)
    ha_ref[...] = h[:, :DH]
    hb_ref[...] = h[:, DH:]
    s_ref[...] = jnp.dot(h, a2_ref[...], preferred_element_type=jnp.float32)
    rp = jnp.dot(rel_ref[...], wr_ref[...], preferred_element_type=jnp.float32)
    re_ref[...] = jnp.dot(rp, ae_ref[...], preferred_element_type=jnp.float32)


_tc_proj = pl.pallas_call(
    _tc_proj_body,
    out_shape=[
        jax.ShapeDtypeStruct((N, DH), jnp.float32),
        jax.ShapeDtypeStruct((N, DH), jnp.float32),
        jax.ShapeDtypeStruct((N, 8), jnp.float32),
        jax.ShapeDtypeStruct((NUM_ET, 8), jnp.float32),
    ],
)


@functools.partial(
    pl.kernel,
    out_type=[
        jax.ShapeDtypeStruct((E,), jnp.float32),
        jax.ShapeDtypeStruct((NC, NPAD), jnp.float32),
    ],
    mesh=_mesh,
    compiler_params=_sc_params,
    scratch_types=[
        pltpu.VMEM((EPT,), jnp.int32),
        pltpu.VMEM((NCHUNK, C), jnp.int32),
        pltpu.VMEM((EPT,), jnp.int32),
        pltpu.VMEM((NPAD,), jnp.float32),
        pltpu.VMEM((NPAD,), jnp.float32),
        pltpu.VMEM((ET_PAD,), jnp.float32),
        pltpu.VMEM((EPT,), jnp.float32),
        pltpu.VMEM_SHARED((NPAD,), jnp.float32),
        pltpu.SemaphoreType.DMA,
    ],
)
def _sc_scores(src_hbm, dstr_hbm, et_hbm, hl_hbm, hr_hbm, re_hbm,
               ex_hbm, psum_hbm,
               src_v, dst_v, et_v, hl_v, hr_v, re_v, ex_v, ssum_sh, sem):
    cid = lax.axis_index("c")
    sid = lax.axis_index("s")
    w = sid * NC + cid
    base = pl.multiple_of(w * EPT, 8)
    pltpu.sync_copy(src_hbm.at[pl.ds(base, EPT)], src_v)
    pltpu.sync_copy(dstr_hbm.at[w], dst_v)
    pltpu.sync_copy(et_hbm.at[pl.ds(base, EPT)], et_v)
    pltpu.sync_copy(hl_hbm, hl_v)
    pltpu.sync_copy(hr_hbm, hr_v)
    pltpu.sync_copy(re_hbm, re_v)

    def zbody(i, carry):
        ex_v[pl.ds(i * L, L)] = jnp.zeros((L,), jnp.float32)
        return carry

    lax.fori_loop(0, SLC // L, zbody, 0)
    soff = pl.multiple_of(sid * SLC, 8)
    pltpu.sync_copy(ex_v.at[pl.ds(0, SLC)], ssum_sh.at[pl.ds(soff, SLC)])
    plsc.subcore_barrier()

    def cbody(c, carry):
        for j in range(VPC):
            sl = pl.ds(c * C + j * L, L)
            sv = src_v[sl]
            dv = dst_v[c, pl.ds(j * L, L)]
            ev = et_v[sl]
            gl = plsc.load_gather(hl_v, [sv])
            gr = plsc.load_gather(hr_v, [dv])
            ge = plsc.load_gather(re_v, [ev])
            z = gl + gr + ge
            e = jnp.where(z >= 0.0, z, 0.2 * z)
            ex_v[sl] = jnp.exp(e)
        return carry

    lax.fori_loop(0, NCHUNK, cbody, 0)

    def fire(c, carry):
        pltpu.async_copy(
            ex_v.at[pl.ds(pl.multiple_of(c * C, 8), C)],
            ssum_sh.at[dst_v.at[c]], sem, add=True)
        return carry

    lax.fori_loop(0, NCHUNK, fire, 0)

    def drain(c, carry):
        pltpu.make_async_copy(
            ex_v.at[pl.ds(0, C)], ssum_sh.at[dst_v.at[0]], sem).wait()
        return carry

    lax.fori_loop(0, NCHUNK, drain, 0)
    plsc.subcore_barrier()

    pltpu.sync_copy(ex_v, ex_hbm.at[pl.ds(base, EPT)])
    pltpu.sync_copy(ssum_sh.at[pl.ds(soff, SLC)],
                    psum_hbm.at[cid, pl.ds(soff, SLC)])


@functools.partial(
    pl.kernel,
    out_type=jax.ShapeDtypeStruct((E,), jnp.float32),
    mesh=_mesh,
    compiler_params=_sc_params,
    scratch_types=[
        pltpu.VMEM((EPT,), jnp.int32),
        pltpu.VMEM((EPT,), jnp.float32),
        pltpu.VMEM((NPAD,), jnp.float32),
        pltpu.VMEM((NPAD,), jnp.float32),
    ],
)
def _sc_alpha(dst_hbm, ex_hbm, psum_hbm, alpha_hbm,
              dst_v, ex_v, s0_v, s1_v):
    w = lax.axis_index("s") * NC + lax.axis_index("c")
    base = pl.multiple_of(w * EPT, 8)
    pltpu.sync_copy(dst_hbm.at[pl.ds(base, EPT)], dst_v)
    pltpu.sync_copy(ex_hbm.at[pl.ds(base, EPT)], ex_v)
    pltpu.sync_copy(psum_hbm.at[0], s0_v)
    pltpu.sync_copy(psum_hbm.at[1], s1_v)

    def addbody(i, carry):
        sl = pl.ds(i * L, L)
        s0_v[sl] = s0_v[sl] + s1_v[sl]
        return carry

    lax.fori_loop(0, NPAD // L, addbody, 0)

    def body(i, carry):
        sl = pl.ds(i * L, L)
        dv = dst_v[sl]
        s = plsc.load_gather(s0_v, [dv])
        ex_v[sl] = ex_v[sl] / (s + 1e-16)
        return carry

    lax.fori_loop(0, EPT // L, body, 0)
    pltpu.sync_copy(ex_v, alpha_hbm.at[pl.ds(base, EPT)])


@functools.partial(
    pl.kernel,
    out_type=jax.ShapeDtypeStruct((NC, NPAD, DH), jnp.float32),
    mesh=_mesh,
    compiler_params=_sc_params,
    scratch_types=[
        pltpu.VMEM((EPT,), jnp.int32),
        pltpu.VMEM((NCHUNK, C), jnp.int32),
        pltpu.VMEM((EPT,), jnp.float32),
        pltpu.VMEM((C, DH), jnp.float32),
        pltpu.VMEM((C, DH), jnp.float32),
        pltpu.VMEM((C, DH), jnp.float32),
        pltpu.VMEM((C, DH), jnp.float32),
        pltpu.VMEM_SHARED((NPAD, DH), jnp.float32),
        pltpu.SemaphoreType.DMA,
        pltpu.SemaphoreType.DMA,
        pltpu.SemaphoreType.DMA,
        pltpu.SemaphoreType.DMA,
    ],
)
def _sc_msg(src_hbm, dstr_hbm, al_hbm, h_hbm, agg_hbm,
            src_v, dst_v, al_v, rows0_v, rows1_v, sb0_v, sb1_v, acc_sh,
            sem_g0, sem_g1, sem_s0, sem_s1):
    cid = lax.axis_index("c")
    sid = lax.axis_index("s")
    w = sid * NC + cid
    base = pl.multiple_of(w * EPT, 8)

    rows_bufs = (rows0_v, rows1_v)
    sbufs = (sb0_v, sb1_v)
    gsems = (sem_g0, sem_g1)
    ssems = (sem_s0, sem_s1)

    pltpu.sync_copy(src_hbm.at[pl.ds(base, EPT)], src_v)
    pltpu.sync_copy(dstr_hbm.at[w], dst_v)
    pltpu.sync_copy(al_hbm.at[pl.ds(base, EPT)], al_v)

    def zrow(i, carry):
        sb0_v[i // (DH // L), pl.ds((i % (DH // L)) * L, L)] = (
            jnp.zeros((L,), jnp.float32))
        return carry

    lax.fori_loop(0, C * DH // L, zrow, 0)
    rbase = pl.multiple_of(sid * SLC, 8)
    for t in range(SLC // C):
        pltpu.sync_copy(sb0_v, acc_sh.at[pl.ds(rbase + t * C, C), :])
    plsc.subcore_barrier()

    def start_gather(c, b):
        pltpu.async_copy(
            h_hbm.at[src_v.at[pl.ds(c * C, C)]], rows_bufs[b], gsems[b])

    def start_scatter(c, b):
        pltpu.async_copy(
            sbufs[b], acc_sh.at[dst_v.at[c]], ssems[b], add=True)

    def wait_gather(b):
        pltpu.make_async_copy(
            h_hbm.at[src_v.at[pl.ds(0, C)]], rows_bufs[b], gsems[b]).wait()

    def wait_scatter(b):
        pltpu.make_async_copy(
            sbufs[b], acc_sh.at[dst_v.at[0]], ssems[b]).wait()

    def compute(c, b):
        rows = rows_bufs[b]
        sb = sbufs[b]

        def ebody(e, carry):
            av = plsc.load_gather(al_v, [lax.broadcast(c * C + e, (L,))])
            for k in range(DH // L):
                sl = pl.ds(k * L, L)
                sb[e, sl] = rows[e, sl] * av
            return carry

        lax.fori_loop(0, C, ebody, 0)

    start_gather(0, 0)
    start_gather(1, 1)

    def gbody(g, carry):
        for b in range(2):
            c = 2 * g + b
            wait_gather(b)

            @pl.when(g >= 1)
            def _():
                wait_scatter(b)

            compute(c, b)

            @pl.when(c + 2 < NCHUNK)
            def _():
                start_gather(c + 2, b)

            start_scatter(c, b)
        return carry

    lax.fori_loop(0, (NCHUNK - 1) // 2, gbody, 0)

    ct = NCHUNK - 1
    wait_gather(0)
    wait_scatter(0)
    compute(ct, 0)
    start_scatter(ct, 0)
    wait_scatter(1)
    wait_scatter(0)

    plsc.subcore_barrier()
    pltpu.sync_copy(acc_sh.at[pl.ds(rbase, SLC), :],
                    agg_hbm.at[cid, pl.ds(rbase, SLC), :])


def _tc_selu_body(a_ref, b_ref, o_ref):
    xa = a_ref[0] + a_ref[1]
    xb = b_ref[0] + b_ref[1]
    x = jnp.concatenate([xa, xb], axis=-1)
    o_ref[...] = _SELU_LAM * jnp.where(
        x > 0.0, x, _SELU_ALPHA * (jnp.exp(x) - 1.0))


_tc_selu = pl.pallas_call(
    _tc_selu_body,
    out_shape=jax.ShapeDtypeStruct((NPAD, D), jnp.float32),
)


def kernel(node_features, edge_index, edge_type, W, W_r, rel_emb,
           a_l, a_r, a_e):
    src = edge_index[0].astype(jnp.int32)
    dst = edge_index[1].astype(jnp.int32)
    et = edge_type.astype(jnp.int32)
    a2 = jnp.pad(jnp.stack([a_l, a_r], axis=1), ((0, 0), (0, 6)))
    ae8 = jnp.pad(a_e[:, None], ((0, 0), (0, 7)))

    ha, hb, s8, re8 = _tc_proj(node_features, W, a2, W_r, rel_emb, ae8)
    hl = jnp.pad(s8[:, 0], (0, NPAD - N))
    hr = jnp.pad(s8[:, 1], (0, NPAD - N))
    re64 = jnp.pad(re8[:, 0], (0, ET_PAD - NUM_ET))

    dst_r = dst.reshape(NW, NCHUNK, C)
    ex, psum = _sc_scores(src, dst_r, et, hl, hr, re64)
    alpha = _sc_alpha(dst, ex, psum)
    agg_a = _sc_msg(src, dst_r, alpha, ha)
    agg_b = _sc_msg(src, dst_r, alpha, hb)
    out = _tc_selu(agg_a, agg_b)
    return out[:N]


# speedup vs baseline: 26.3678x; 1.0577x over previous
import functools

import jax
import jax.numpy as jnp
from jax import lax
from jax.experimental import pallas as pl
from jax.experimental.pallas import tpu as pltpu
from jax.experimental.pallas import tpu_sc as plsc

N = 10000
NPAD = 10240
E = 320000
D = 128
DH = 64
NUM_ET = 40
ET_PAD = 128

L = 16
NC = 2
NS = 16
NW = NC * NS
EPT = E // NW

C = 80
NCHUNK = EPT // C
VPC = C // L
UN = 8

SLC = NPAD // NS

_SELU_LAM = 1.0507009873554804934193349852946
_SELU_ALPHA = 1.6732632423543772848170429916717

_mesh = plsc.VectorSubcoreMesh(core_axis_name="c", subcore_axis_name="s")
_sc_params = pltpu.CompilerParams(
    needs_layout_passes=False, use_tc_tiling_on_sc=False)


def _tc_proj_body(x_ref, w_ref, a2_ref, wr_ref, rel_ref, ae_ref,
                  ha_ref, hb_ref, s_ref, re_ref):
    h = jnp.dot(x_ref[...], w_ref[...], preferred_element_type=jnp.float32)
    ha_ref[...] = h[:, :DH]
    hb_ref[...] = h[:, DH:]
    s_ref[...] = jnp.dot(h, a2_ref[...], preferred_element_type=jnp.float32)
    rp = jnp.dot(rel_ref[...], wr_ref[...], preferred_element_type=jnp.float32)
    re_ref[...] = jnp.dot(rp, ae_ref[...], preferred_element_type=jnp.float32)


_tc_proj = pl.pallas_call(
    _tc_proj_body,
    out_shape=[
        jax.ShapeDtypeStruct((N, DH), jnp.float32),
        jax.ShapeDtypeStruct((N, DH), jnp.float32),
        jax.ShapeDtypeStruct((N, 8), jnp.float32),
        jax.ShapeDtypeStruct((NUM_ET, 8), jnp.float32),
    ],
)


def _zero_vmem_rows(buf, nvecs):
    kv = DH // L

    def zbody(i, carry):
        buf[i // kv, pl.ds((i % kv) * L, L)] = jnp.zeros((L,), jnp.float32)
        return carry

    lax.fori_loop(0, nvecs, zbody, 0)


def _msg_pipeline(src_v, dst_v, et_v, hl_v, hr_v, re_v, ex_v,
                  rows_bufs, sbufs, gsems, ssems, h_hbm, acc_sh,
                  psum_sh=None, sem_p=None):
    def start_gather(c, b):
        pltpu.async_copy(
            h_hbm.at[src_v.at[pl.ds(c * C, C)]], rows_bufs[b], gsems[b])

    def start_scatter(c, b):
        pltpu.async_copy(
            sbufs[b], acc_sh.at[dst_v.at[c]], ssems[b], add=True)

    def wait_gather(b):
        pltpu.make_async_copy(
            h_hbm.at[src_v.at[pl.ds(0, C)]], rows_bufs[b], gsems[b]).wait()

    def wait_scatter(b):
        pltpu.make_async_copy(
            sbufs[b], acc_sh.at[dst_v.at[0]], ssems[b]).wait()

    def compute(c, b):
        rows = rows_bufs[b]
        sb = sbufs[b]

        for j in range(VPC):
            sl = pl.ds(c * C + j * L, L)
            sv = src_v[sl]
            dv = dst_v[c, pl.ds(j * L, L)]
            ev = et_v[sl]
            z = (plsc.load_gather(hl_v, [sv])
                 + plsc.load_gather(hr_v, [dv])
                 + plsc.load_gather(re_v, [ev]))
            e = jnp.where(z >= 0.0, z, 0.2 * z)
            ex_v[sl] = jnp.exp(e)

        if psum_sh is not None:
            pltpu.async_copy(
                ex_v.at[pl.ds(pl.multiple_of(c * C, 8), C)],
                psum_sh.at[dst_v.at[c]], sem_p, add=True)

        def ebody(q, carry):
            for u in range(UN):
                av = plsc.load_gather(
                    ex_v, [lax.broadcast(c * C + q * UN + u, (L,))])
                for k in range(DH // L):
                    sl = pl.ds(k * L, L)
                    sb[q * UN + u, sl] = rows[q * UN + u, sl] * av
            return carry

        lax.fori_loop(0, C // UN, ebody, 0)

    start_gather(0, 0)
    start_gather(1, 1)

    def gbody(g, carry):
        for b in range(2):
            c = 2 * g + b
            wait_gather(b)

            @pl.when(g >= 1)
            def _():
                wait_scatter(b)

            compute(c, b)

            @pl.when(c + 2 < NCHUNK)
            def _():
                start_gather(c + 2, b)

            start_scatter(c, b)
        return carry

    lax.fori_loop(0, (NCHUNK - 1) // 2, gbody, 0)

    ct = NCHUNK - 1
    wait_gather(0)
    wait_scatter(0)
    compute(ct, 0)
    start_scatter(ct, 0)
    wait_scatter(1)
    wait_scatter(0)


_msg_scratch = [
    pltpu.VMEM((EPT,), jnp.int32),
    pltpu.VMEM((NCHUNK, C), jnp.int32),
    pltpu.VMEM((EPT,), jnp.int32),
    pltpu.VMEM((NPAD,), jnp.float32),
    pltpu.VMEM((NPAD,), jnp.float32),
    pltpu.VMEM((ET_PAD,), jnp.float32),
    pltpu.VMEM((EPT,), jnp.float32),
    pltpu.VMEM((C, DH), jnp.float32),
    pltpu.VMEM((C, DH), jnp.float32),
    pltpu.VMEM((C, DH), jnp.float32),
    pltpu.VMEM((C, DH), jnp.float32),
    pltpu.VMEM_SHARED((NPAD, DH), jnp.float32),
    pltpu.SemaphoreType.DMA,
    pltpu.SemaphoreType.DMA,
    pltpu.SemaphoreType.DMA,
    pltpu.SemaphoreType.DMA,
]


@functools.partial(
    pl.kernel,
    out_type=[
        jax.ShapeDtypeStruct((NC, NPAD, DH), jnp.float32),
        jax.ShapeDtypeStruct((NC, NPAD), jnp.float32),
    ],
    mesh=_mesh,
    compiler_params=_sc_params,
    scratch_types=_msg_scratch + [
        pltpu.VMEM_SHARED((NPAD,), jnp.float32),
        pltpu.SemaphoreType.DMA,
    ],
)
def _sc_msg_a(src_hbm, dstr_hbm, et_hbm, hl_hbm, hr_hbm, re_hbm, h_hbm,
              agg_hbm, psum_hbm,
              src_v, dst_v, et_v, hl_v, hr_v, re_v, ex_v,
              rows0_v, rows1_v, sb0_v, sb1_v, acc_sh,
              sem_g0, sem_g1, sem_s0, sem_s1, psum_sh, sem_p):
    cid = lax.axis_index("c")
    sid = lax.axis_index("s")
    w = sid * NC + cid
    base = pl.multiple_of(w * EPT, 8)

    pltpu.sync_copy(src_hbm.at[pl.ds(base, EPT)], src_v)
    pltpu.sync_copy(dstr_hbm.at[w], dst_v)
    pltpu.sync_copy(et_hbm.at[pl.ds(base, EPT)], et_v)
    pltpu.sync_copy(hl_hbm, hl_v)
    pltpu.sync_copy(hr_hbm, hr_v)
    pltpu.sync_copy(re_hbm, re_v)

    _zero_vmem_rows(sb0_v, C * DH // L)
    rbase = pl.multiple_of(sid * SLC, 8)
    for t in range(SLC // C):
        pltpu.sync_copy(sb0_v, acc_sh.at[pl.ds(rbase + t * C, C), :])

    def zex(i, carry):
        ex_v[pl.ds(i * L, L)] = jnp.zeros((L,), jnp.float32)
        return carry

    lax.fori_loop(0, SLC // L, zex, 0)
    pltpu.sync_copy(ex_v.at[pl.ds(0, SLC)], psum_sh.at[pl.ds(rbase, SLC)])
    plsc.subcore_barrier()

    _msg_pipeline(src_v, dst_v, et_v, hl_v, hr_v, re_v, ex_v,
                  (rows0_v, rows1_v), (sb0_v, sb1_v),
                  (sem_g0, sem_g1), (sem_s0, sem_s1), h_hbm, acc_sh,
                  psum_sh=psum_sh, sem_p=sem_p)

    def drain_p(c, carry):
        pltpu.make_async_copy(
            ex_v.at[pl.ds(0, C)], psum_sh.at[dst_v.at[0]], sem_p).wait()
        return carry

    lax.fori_loop(0, NCHUNK, drain_p, 0)
    plsc.subcore_barrier()

    pltpu.sync_copy(acc_sh.at[pl.ds(rbase, SLC), :],
                    agg_hbm.at[cid, pl.ds(rbase, SLC), :])
    pltpu.sync_copy(psum_sh.at[pl.ds(rbase, SLC)],
                    psum_hbm.at[cid, pl.ds(rbase, SLC)])


@functools.partial(
    pl.kernel,
    out_type=jax.ShapeDtypeStruct((NC, NPAD, DH), jnp.float32),
    mesh=_mesh,
    compiler_params=_sc_params,
    scratch_types=_msg_scratch,
)
def _sc_msg_b(src_hbm, dstr_hbm, et_hbm, hl_hbm, hr_hbm, re_hbm, h_hbm,
              agg_hbm,
              src_v, dst_v, et_v, hl_v, hr_v, re_v, ex_v,
              rows0_v, rows1_v, sb0_v, sb1_v, acc_sh,
              sem_g0, sem_g1, sem_s0, sem_s1):
    cid = lax.axis_index("c")
    sid = lax.axis_index("s")
    w = sid * NC + cid
    base = pl.multiple_of(w * EPT, 8)

    pltpu.sync_copy(src_hbm.at[pl.ds(base, EPT)], src_v)
    pltpu.sync_copy(dstr_hbm.at[w], dst_v)
    pltpu.sync_copy(et_hbm.at[pl.ds(base, EPT)], et_v)
    pltpu.sync_copy(hl_hbm, hl_v)
    pltpu.sync_copy(hr_hbm, hr_v)
    pltpu.sync_copy(re_hbm, re_v)

    _zero_vmem_rows(sb0_v, C * DH // L)
    rbase = pl.multiple_of(sid * SLC, 8)
    for t in range(SLC // C):
        pltpu.sync_copy(sb0_v, acc_sh.at[pl.ds(rbase + t * C, C), :])
    plsc.subcore_barrier()

    _msg_pipeline(src_v, dst_v, et_v, hl_v, hr_v, re_v, ex_v,
                  (rows0_v, rows1_v), (sb0_v, sb1_v),
                  (sem_g0, sem_g1), (sem_s0, sem_s1), h_hbm, acc_sh)

    plsc.subcore_barrier()
    pltpu.sync_copy(acc_sh.at[pl.ds(rbase, SLC), :],
                    agg_hbm.at[cid, pl.ds(rbase, SLC), :])


def _tc_fin_body(a_ref, b_ref, p_ref, o_ref):
    s = p_ref[0] + p_ref[1] + 1e-16
    x = jnp.concatenate([a_ref[0] + a_ref[1], b_ref[0] + b_ref[1]], axis=-1)
    x = x / s[:, None]
    o_ref[...] = _SELU_LAM * jnp.where(
        x > 0.0, x, _SELU_ALPHA * (jnp.exp(x) - 1.0))


_tc_fin = pl.pallas_call(
    _tc_fin_body,
    out_shape=jax.ShapeDtypeStruct((NPAD, D), jnp.float32),
)


def kernel(node_features, edge_index, edge_type, W, W_r, rel_emb,
           a_l, a_r, a_e):
    src = edge_index[0].astype(jnp.int32)
    dst = edge_index[1].astype(jnp.int32)
    et = edge_type.astype(jnp.int32)
    a2 = jnp.pad(jnp.stack([a_l, a_r], axis=1), ((0, 0), (0, 6)))
    ae8 = jnp.pad(a_e[:, None], ((0, 0), (0, 7)))

    ha, hb, s8, re8 = _tc_proj(node_features, W, a2, W_r, rel_emb, ae8)
    hl = jnp.pad(s8[:, 0], (0, NPAD - N))
    hr = jnp.pad(s8[:, 1], (0, NPAD - N))
    re64 = jnp.pad(re8[:, 0], (0, ET_PAD - NUM_ET))

    dst_r = dst.reshape(NW, NCHUNK, C)
    agg_a, psum = _sc_msg_a(src, dst_r, et, hl, hr, re64, ha)
    agg_b = _sc_msg_b(src, dst_r, et, hl, hr, re64, hb)
    out = _tc_fin(agg_a, agg_b, psum)
    return out[:N]


# speedup vs baseline: 30.4462x; 1.1547x over previous
import functools

import jax
import jax.numpy as jnp
from jax import lax
from jax.experimental import pallas as pl
from jax.experimental.pallas import tpu as pltpu
from jax.experimental.pallas import tpu_sc as plsc

N = 10000
NPAD = 10240
E = 320000
D = 128
DH = 64
NUM_ET = 40
ET_PAD = 128

L = 16
NC = 2
NS = 16
NW = NC * NS
EPT = E // NW

C = 80
NCHUNK = EPT // C
VPC = C // L
UN = 8

SLC = NPAD // NS

_SELU_LAM = 1.0507009873554804934193349852946
_SELU_ALPHA = 1.6732632423543772848170429916717

_mesh = plsc.VectorSubcoreMesh(core_axis_name="c", subcore_axis_name="s")
_sc_params = pltpu.CompilerParams(
    needs_layout_passes=False, use_tc_tiling_on_sc=False)


def _tc_proj_body(x_ref, w_ref, a2_ref, wr_ref, rel_ref, ae_ref,
                  ha_ref, hb_ref, s_ref, re_ref):
    h = jnp.dot(x_ref[...], w_ref[...], preferred_element_type=jnp.float32)
    ha_ref[...] = h[:, :DH]
    hb_ref[...] = h[:, DH:]
    s_ref[...] = jnp.dot(h, a2_ref[...], preferred_element_type=jnp.float32)
    rp = jnp.dot(rel_ref[...], wr_ref[...], preferred_element_type=jnp.float32)
    re_ref[...] = jnp.dot(rp, ae_ref[...], preferred_element_type=jnp.float32)


_tc_proj = pl.pallas_call(
    _tc_proj_body,
    out_shape=[
        jax.ShapeDtypeStruct((N, DH), jnp.float32),
        jax.ShapeDtypeStruct((N, DH), jnp.float32),
        jax.ShapeDtypeStruct((N, 8), jnp.float32),
        jax.ShapeDtypeStruct((NUM_ET, 8), jnp.float32),
    ],
)


def _zero_vmem_rows(buf, nvecs):
    kv = DH // L

    def zbody(i, carry):
        buf[i // kv, pl.ds((i % kv) * L, L)] = jnp.zeros((L,), jnp.float32)
        return carry

    lax.fori_loop(0, nvecs, zbody, 0)


def _msg_pipeline(src_v, dst_v, et_v, hl_v, hr_v, re_v, ex_v,
                  rows_bufs, sbufs, gsems, ssems, h_hbm, acc_sh,
                  psum_sh=None, sem_p=None):
    def start_gather(c, b):
        pltpu.async_copy(
            h_hbm.at[src_v.at[pl.ds(c * C, C)]], rows_bufs[b], gsems[b])

    def start_scatter(c, b):
        pltpu.async_copy(
            sbufs[b], acc_sh.at[dst_v.at[c]], ssems[b], add=True)

    def wait_gather(b):
        pltpu.make_async_copy(
            h_hbm.at[src_v.at[pl.ds(0, C)]], rows_bufs[b], gsems[b]).wait()

    def wait_scatter(b):
        pltpu.make_async_copy(
            sbufs[b], acc_sh.at[dst_v.at[0]], ssems[b]).wait()

    def compute(c, b):
        rows = rows_bufs[b]
        sb = sbufs[b]

        for j in range(VPC):
            sl = pl.ds(c * C + j * L, L)
            sv = src_v[sl]
            dv = dst_v[c, pl.ds(j * L, L)]
            ev = et_v[sl]
            z = (plsc.load_gather(hl_v, [sv])
                 + plsc.load_gather(hr_v, [dv])
                 + plsc.load_gather(re_v, [ev]))
            e = jnp.where(z >= 0.0, z, 0.2 * z)
            ex_v[sl] = jnp.exp(e)

        if psum_sh is not None:
            pltpu.async_copy(
                ex_v.at[pl.ds(pl.multiple_of(c * C, 8), C)],
                psum_sh.at[dst_v.at[c]], sem_p, add=True)

        @plsc.parallel_loop(0, C, 1, unroll=UN)
        def _(e):
            av = plsc.load_gather(ex_v, [lax.broadcast(c * C + e, (L,))])
            for k in range(DH // L):
                sl = pl.ds(k * L, L)
                sb[e, sl] = rows[e, sl] * av

    start_gather(0, 0)
    start_gather(1, 1)

    def gbody(g, carry):
        for b in range(2):
            c = 2 * g + b
            wait_gather(b)

            @pl.when(g >= 1)
            def _():
                wait_scatter(b)

            compute(c, b)

            @pl.when(c + 2 < NCHUNK)
            def _():
                start_gather(c + 2, b)

            start_scatter(c, b)
        return carry

    lax.fori_loop(0, (NCHUNK - 1) // 2, gbody, 0)

    ct = NCHUNK - 1
    wait_gather(0)
    wait_scatter(0)
    compute(ct, 0)
    start_scatter(ct, 0)
    wait_scatter(1)
    wait_scatter(0)


_msg_scratch = [
    pltpu.VMEM((EPT,), jnp.int32),
    pltpu.VMEM((NCHUNK, C), jnp.int32),
    pltpu.VMEM((EPT,), jnp.int32),
    pltpu.VMEM((NPAD,), jnp.float32),
    pltpu.VMEM((NPAD,), jnp.float32),
    pltpu.VMEM((ET_PAD,), jnp.float32),
    pltpu.VMEM((EPT,), jnp.float32),
    pltpu.VMEM((C, DH), jnp.float32),
    pltpu.VMEM((C, DH), jnp.float32),
    pltpu.VMEM((C, DH), jnp.float32),
    pltpu.VMEM((C, DH), jnp.float32),
    pltpu.VMEM_SHARED((NPAD, DH), jnp.float32),
    pltpu.SemaphoreType.DMA,
    pltpu.SemaphoreType.DMA,
    pltpu.SemaphoreType.DMA,
    pltpu.SemaphoreType.DMA,
]


@functools.partial(
    pl.kernel,
    out_type=[
        jax.ShapeDtypeStruct((NC, NPAD, DH), jnp.float32),
        jax.ShapeDtypeStruct((NC, NPAD), jnp.float32),
    ],
    mesh=_mesh,
    compiler_params=_sc_params,
    scratch_types=_msg_scratch + [
        pltpu.VMEM_SHARED((NPAD,), jnp.float32),
        pltpu.SemaphoreType.DMA,
    ],
)
def _sc_msg_a(src_hbm, dstr_hbm, et_hbm, hl_hbm, hr_hbm, re_hbm, h_hbm,
              agg_hbm, psum_hbm,
              src_v, dst_v, et_v, hl_v, hr_v, re_v, ex_v,
              rows0_v, rows1_v, sb0_v, sb1_v, acc_sh,
              sem_g0, sem_g1, sem_s0, sem_s1, psum_sh, sem_p):
    cid = lax.axis_index("c")
    sid = lax.axis_index("s")
    w = sid * NC + cid
    base = pl.multiple_of(w * EPT, 8)

    pltpu.sync_copy(src_hbm.at[pl.ds(base, EPT)], src_v)
    pltpu.sync_copy(dstr_hbm.at[w], dst_v)
    pltpu.sync_copy(et_hbm.at[pl.ds(base, EPT)], et_v)
    pltpu.sync_copy(hl_hbm, hl_v)
    pltpu.sync_copy(hr_hbm, hr_v)
    pltpu.sync_copy(re_hbm, re_v)

    _zero_vmem_rows(sb0_v, C * DH // L)
    rbase = pl.multiple_of(sid * SLC, 8)
    for t in range(SLC // C):
        pltpu.sync_copy(sb0_v, acc_sh.at[pl.ds(rbase + t * C, C), :])

    def zex(i, carry):
        ex_v[pl.ds(i * L, L)] = jnp.zeros((L,), jnp.float32)
        return carry

    lax.fori_loop(0, SLC // L, zex, 0)
    pltpu.sync_copy(ex_v.at[pl.ds(0, SLC)], psum_sh.at[pl.ds(rbase, SLC)])
    plsc.subcore_barrier()

    _msg_pipeline(src_v, dst_v, et_v, hl_v, hr_v, re_v, ex_v,
                  (rows0_v, rows1_v), (sb0_v, sb1_v),
                  (sem_g0, sem_g1), (sem_s0, sem_s1), h_hbm, acc_sh,
                  psum_sh=psum_sh, sem_p=sem_p)

    def drain_p(c, carry):
        pltpu.make_async_copy(
            ex_v.at[pl.ds(0, C)], psum_sh.at[dst_v.at[0]], sem_p).wait()
        return carry

    lax.fori_loop(0, NCHUNK, drain_p, 0)
    plsc.subcore_barrier()

    pltpu.sync_copy(acc_sh.at[pl.ds(rbase, SLC), :],
                    agg_hbm.at[cid, pl.ds(rbase, SLC), :])
    pltpu.sync_copy(psum_sh.at[pl.ds(rbase, SLC)],
                    psum_hbm.at[cid, pl.ds(rbase, SLC)])


@functools.partial(
    pl.kernel,
    out_type=jax.ShapeDtypeStruct((NC, NPAD, DH), jnp.float32),
    mesh=_mesh,
    compiler_params=_sc_params,
    scratch_types=_msg_scratch,
)
def _sc_msg_b(src_hbm, dstr_hbm, et_hbm, hl_hbm, hr_hbm, re_hbm, h_hbm,
              agg_hbm,
              src_v, dst_v, et_v, hl_v, hr_v, re_v, ex_v,
              rows0_v, rows1_v, sb0_v, sb1_v, acc_sh,
              sem_g0, sem_g1, sem_s0, sem_s1):
    cid = lax.axis_index("c")
    sid = lax.axis_index("s")
    w = sid * NC + cid
    base = pl.multiple_of(w * EPT, 8)

    pltpu.sync_copy(src_hbm.at[pl.ds(base, EPT)], src_v)
    pltpu.sync_copy(dstr_hbm.at[w], dst_v)
    pltpu.sync_copy(et_hbm.at[pl.ds(base, EPT)], et_v)
    pltpu.sync_copy(hl_hbm, hl_v)
    pltpu.sync_copy(hr_hbm, hr_v)
    pltpu.sync_copy(re_hbm, re_v)

    _zero_vmem_rows(sb0_v, C * DH // L)
    rbase = pl.multiple_of(sid * SLC, 8)
    for t in range(SLC // C):
        pltpu.sync_copy(sb0_v, acc_sh.at[pl.ds(rbase + t * C, C), :])
    plsc.subcore_barrier()

    _msg_pipeline(src_v, dst_v, et_v, hl_v, hr_v, re_v, ex_v,
                  (rows0_v, rows1_v), (sb0_v, sb1_v),
                  (sem_g0, sem_g1), (sem_s0, sem_s1), h_hbm, acc_sh)

    plsc.subcore_barrier()
    pltpu.sync_copy(acc_sh.at[pl.ds(rbase, SLC), :],
                    agg_hbm.at[cid, pl.ds(rbase, SLC), :])


def _tc_fin_body(a_ref, b_ref, p_ref, o_ref):
    s = p_ref[0] + p_ref[1] + 1e-16
    x = jnp.concatenate([a_ref[0] + a_ref[1], b_ref[0] + b_ref[1]], axis=-1)
    x = x / s[:, None]
    o_ref[...] = _SELU_LAM * jnp.where(
        x > 0.0, x, _SELU_ALPHA * (jnp.exp(x) - 1.0))


_tc_fin = pl.pallas_call(
    _tc_fin_body,
    out_shape=jax.ShapeDtypeStruct((NPAD, D), jnp.float32),
)


def kernel(node_features, edge_index, edge_type, W, W_r, rel_emb,
           a_l, a_r, a_e):
    src = edge_index[0].astype(jnp.int32)
    dst = edge_index[1].astype(jnp.int32)
    et = edge_type.astype(jnp.int32)
    a2 = jnp.pad(jnp.stack([a_l, a_r], axis=1), ((0, 0), (0, 6)))
    ae8 = jnp.pad(a_e[:, None], ((0, 0), (0, 7)))

    ha, hb, s8, re8 = _tc_proj(node_features, W, a2, W_r, rel_emb, ae8)
    hl = jnp.pad(s8[:, 0], (0, NPAD - N))
    hr = jnp.pad(s8[:, 1], (0, NPAD - N))
    re64 = jnp.pad(re8[:, 0], (0, ET_PAD - NUM_ET))

    dst_r = dst.reshape(NW, NCHUNK, C)
    agg_a, psum = _sc_msg_a(src, dst_r, et, hl, hr, re64, ha)
    agg_b = _sc_msg_b(src, dst_r, et, hl, hr, re64, hb)
    out = _tc_fin(agg_a, agg_b, psum)
    return out[:N]
